# baseline (speedup 1.0000x reference)
import functools

import jax
import jax.numpy as jnp
from jax import lax
from jax.experimental import pallas as pl
from jax.experimental.pallas import tpu as pltpu
from jax.experimental.pallas import tpu_sc as plsc

_CHUNK = 128


@functools.lru_cache(maxsize=None)
def _build(B, V, D):
    info = plsc.get_sparse_core_info()
    NC, NS = info.num_cores, info.num_subcores
    NW = NC * NS
    assert B % (NW * _CHUNK) == 0
    b_per_w = B // NW
    n_ch = b_per_w // _CHUNK
    mesh = plsc.VectorSubcoreMesh(core_axis_name="c", subcore_axis_name="s")

    @functools.partial(
        pl.kernel,
        mesh=mesh,
        out_type=(
            jax.ShapeDtypeStruct((B, D), jnp.float32),
            jax.ShapeDtypeStruct((B, D), jnp.float32),
        ),
        scratch_types=[
            pltpu.VMEM((n_ch, _CHUNK), jnp.int32),
            pltpu.VMEM((n_ch, _CHUNK), jnp.int32),
            pltpu.VMEM((b_per_w, D), jnp.float32),
            pltpu.VMEM((b_per_w, D), jnp.float32),
            pltpu.SemaphoreType.DMA,
            pltpu.SemaphoreType.DMA,
            pltpu.SemaphoreType.DMA,
            pltpu.SemaphoreType.DMA,
        ],
        compiler_params=pltpu.CompilerParams(use_tc_tiling_on_sc=False),
    )
    def k(t_hbm, c_hbm, ww_hbm, wo_hbm, o1_hbm, o2_hbm,
          i1_v, i2_v, r1_v, r2_v, s1, s2, so1, so2):
        wid = lax.axis_index("s") * NC + lax.axis_index("c")
        base = wid * b_per_w
        for j in range(n_ch):
            pltpu.sync_copy(t_hbm.at[pl.ds(base + j * _CHUNK, _CHUNK)], i1_v.at[j])
            pltpu.sync_copy(c_hbm.at[pl.ds(base + j * _CHUNK, _CHUNK)], i2_v.at[j])
        g1 = [pltpu.async_copy(ww_hbm.at[i1_v.at[j]],
                               r1_v.at[pl.ds(j * _CHUNK, _CHUNK)], s1)
              for j in range(n_ch)]
        g2 = [pltpu.async_copy(wo_hbm.at[i2_v.at[j]],
                               r2_v.at[pl.ds(j * _CHUNK, _CHUNK)], s2)
              for j in range(n_ch)]
        for g in g1:
            g.wait()
        w1 = pltpu.async_copy(r1_v, o1_hbm.at[pl.ds(base, b_per_w)], so1)
        for g in g2:
            g.wait()
        w2 = pltpu.async_copy(r2_v, o2_hbm.at[pl.ds(base, b_per_w)], so2)
        w1.wait()
        w2.wait()

    return k


def kernel(target, context, W_word, W_out):
    B = target.shape[0]
    V, D = W_word.shape
    return _build(B, V, D)(target, context, W_word, W_out)
